# baseline (speedup 1.0000x reference)
import jax
import jax.numpy as jnp
from jax.experimental import pallas as pl

_K1 = 11


def _sqnorm_rows(a8):
    return (a8[:, 0:1] * a8[:, 0:1] + a8[:, 1:2] * a8[:, 1:2]
            + a8[:, 2:3] * a8[:, 2:3])


def _sqnorm_cols(bt):
    return (bt[0:1, :] * bt[0:1, :] + bt[1:2, :] * bt[1:2, :]
            + bt[2:3, :] * bt[2:3, :])


def _row_block_size(n):
    return n if n < 1024 else 1024


def _loss_kernel(coarse_ref, fine1_ref, fine2_ref, partial_ref,
                 gt_t_ref, f2_t_ref, out_ref):
    gt_t = gt_t_ref[0]
    f2_t = f2_t_ref[0]
    gt_bfm2 = (gt_t * -2.0).astype(jnp.bfloat16)
    f2_bfm2 = (f2_t * -2.0).astype(jnp.bfloat16)
    b2_gt = _sqnorm_cols(gt_t)
    b2_f2 = _sqnorm_cols(f2_t)
    m_gt = gt_t.shape[1]
    m_f2 = f2_t.shape[1]

    def chamfer(a_ref, bt_bfm2, b2, m):
        n = a_ref.shape[1]
        r = _row_block_size(n)

        def body(i, carry):
            sv, colmin = carry
            a8 = a_ref[0, pl.ds(i * r, r), :]
            a2 = _sqnorm_rows(a8)
            ab2 = jnp.dot(a8.astype(jnp.bfloat16), bt_bfm2,
                          preferred_element_type=jnp.float32)
            e = b2 + ab2
            rmin = jnp.min(e, axis=1, keepdims=True) + a2
            sv = sv + jnp.maximum(rmin, 0.0)
            colmin = jnp.minimum(colmin, jnp.min(a2 + e, axis=0))
            return sv, colmin

        init = (jnp.zeros((r, 1), jnp.float32),
                jnp.full((m,), 1e30, jnp.float32))
        sv, colmin = jax.lax.fori_loop(0, n // r, body, init)
        return jnp.sum(sv), jnp.sum(jnp.maximum(colmin, 0.0))

    s_c_row, s_c_col = chamfer(coarse_ref, gt_bfm2, b2_gt, m_gt)
    s_f1_row, s_f1_col = chamfer(fine1_ref, gt_bfm2, b2_gt, m_gt)
    s_f2_row, s_f2_col = chamfer(fine2_ref, gt_bfm2, b2_gt, m_gt)

    n_p = partial_ref.shape[1]
    r_p = _row_block_size(n_p)

    def pbody(i, sv):
        a8 = partial_ref[0, pl.ds(i * r_p, r_p), :]
        a2 = _sqnorm_rows(a8)
        ab2 = jnp.dot(a8.astype(jnp.bfloat16), f2_bfm2,
                      preferred_element_type=jnp.float32)
        rmin = jnp.min(b2_f2 + ab2, axis=1, keepdims=True) + a2
        return sv + jnp.sqrt(jnp.maximum(rmin, 0.0))

    s_partial = jnp.sum(jax.lax.fori_loop(
        0, n_p // r_p, pbody, jnp.zeros((r_p, 1), jnp.float32)))

    b_aug = jnp.concatenate(
        [-2.0 * f2_t, jnp.ones((1, m_f2), jnp.float32), b2_f2,
         jnp.zeros((6, m_f2), jnp.float32)], axis=0)
    b_hi = b_aug.astype(jnp.bfloat16)
    b_lo = (b_aug - b_hi.astype(jnp.float32)).astype(jnp.bfloat16)
    n_f = fine2_ref.shape[1]
    r_f = _row_block_size(n_f)
    kf = float(_K1)

    def sbody(i, sv):
        a8 = fine2_ref[0, pl.ds(i * r_f, r_f), :]
        a2 = _sqnorm_rows(a8)
        ab2 = jnp.dot(a8.astype(jnp.bfloat16), f2_bfm2,
                      preferred_element_type=jnp.float32)
        keyb = (a2 + (b2_f2 + ab2)).astype(jnp.bfloat16)
        a_aug = jnp.concatenate(
            [a8, a2, jnp.ones((r_f, 1), jnp.float32),
             jnp.zeros((r_f, 6), jnp.float32)], axis=1)
        a_hi = a_aug.astype(jnp.bfloat16)
        a_lo = (a_aug - a_hi.astype(jnp.float32)).astype(jnp.bfloat16)
        val_raw = (jnp.dot(a_hi, b_hi, preferred_element_type=jnp.float32)
                   + (jnp.dot(a_hi, b_lo, preferred_element_type=jnp.float32)
                      + jnp.dot(a_lo, b_hi,
                                preferred_element_type=jnp.float32)))
        val = val_raw

        one_b = jnp.bfloat16(1.0)
        zero_b = jnp.bfloat16(0.0)
        big_b = jnp.bfloat16(1e30)

        m1 = jnp.min(keyb, axis=1, keepdims=True)
        le = keyb <= m1
        cnt0 = jnp.sum(jnp.where(le, one_b, zero_b), axis=1, keepdims=True,
                       dtype=jnp.bfloat16).astype(jnp.float32)
        krem = kf - jnp.minimum(cnt0, kf)
        tau = m1
        c_lt = jnp.zeros((r_f, 1), jnp.float32)
        c_eq = cnt0
        k = jnp.where(le, big_b, keyb)
        for p in range(_K1 - 1):
            m = jnp.min(k, axis=1, keepdims=True)
            le = k <= m
            cnt = jnp.sum(jnp.where(le, one_b, zero_b), axis=1, keepdims=True,
                          dtype=jnp.bfloat16).astype(jnp.float32)
            take = jnp.minimum(cnt, krem)
            sel_p = take > 0.0
            tau = jnp.where(sel_p, m, tau)
            c_lt = jnp.where(sel_p, kf - krem, c_lt)
            c_eq = jnp.where(sel_p, cnt, c_eq)
            krem = krem - take
            if p < _K1 - 2:
                k = jnp.where(le, big_b, k)

        frac = ((kf - c_lt) / c_eq).astype(jnp.bfloat16)
        beta = (1.0 / cnt0).astype(jnp.bfloat16)
        w = jnp.where(keyb < tau, one_b, zero_b)
        w = w + jnp.where(keyb == tau, frac, zero_b)
        w = w - jnp.where(keyb == m1, beta, zero_b)
        picked = jnp.sum(w.astype(jnp.float32) * val, axis=1, keepdims=True)
        return sv + picked

    s_smooth = jnp.sum(jax.lax.fori_loop(
        0, n_f // r_f, sbody, jnp.zeros((r_f, 1), jnp.float32)))

    lane = jax.lax.broadcasted_iota(jnp.int32, (1, 128), 1)
    vals = [s_c_row, s_c_col, s_f1_row, s_f1_col,
            s_f2_row, s_f2_col, s_partial, s_smooth]
    out = jnp.zeros((1, 128), jnp.float32)
    for j, v in enumerate(vals):
        out = jnp.where(lane == j, v, out)
    out_ref[...] = out[None]


def kernel(partial, coarse, fine1, fine2, gt):
    b, n_partial, _ = partial.shape
    n_coarse = coarse.shape[1]
    n_fine1 = fine1.shape[1]
    n_fine2 = fine2.shape[1]
    n_gt = gt.shape[1]

    def pad_rows(x):
        return jnp.concatenate(
            [x, jnp.zeros((b, x.shape[1], 5), x.dtype)], axis=2)

    def pad_t(x):
        xt = jnp.transpose(x, (0, 2, 1))
        return jnp.concatenate(
            [xt, jnp.zeros((b, 5, x.shape[1]), x.dtype)], axis=1)

    spec3 = lambda n: pl.BlockSpec((1, n, 8), lambda i: (i, 0, 0))
    spect = lambda n: pl.BlockSpec((1, 8, n), lambda i: (i, 0, 0))

    sums = pl.pallas_call(
        _loss_kernel,
        grid=(b,),
        in_specs=[spec3(n_coarse), spec3(n_fine1), spec3(n_fine2),
                  spec3(n_partial), spect(n_gt), spect(n_fine2)],
        out_specs=pl.BlockSpec((1, 1, 128), lambda i: (i, 0, 0)),
        out_shape=jax.ShapeDtypeStruct((b, 1, 128), jnp.float32),
    )(pad_rows(coarse), pad_rows(fine1), pad_rows(fine2), pad_rows(partial),
      pad_t(gt), pad_t(fine2))
    sums = sums[:, 0, :]

    cd_coarse = jnp.mean(sums[:, 0] / n_coarse + sums[:, 1] / n_gt)
    cd_fine1 = jnp.mean(sums[:, 2] / n_fine1 + sums[:, 3] / n_gt)
    cd_fine2 = jnp.mean(sums[:, 4] / n_fine2 + sums[:, 5] / n_gt)
    partial_loss = jnp.mean(sums[:, 6]) / n_partial
    smooth_loss = jnp.mean(sums[:, 7]) / (n_fine2 * (_K1 - 1))
    total = (cd_coarse + cd_fine1 + cd_fine2 +
             0.5 * partial_loss + 0.1 * smooth_loss)
    return (total, cd_coarse, cd_fine1, cd_fine2, partial_loss, smooth_loss)


# speedup vs baseline: 1.1885x; 1.1885x over previous
import jax
import jax.numpy as jnp
from jax.experimental import pallas as pl

_K1 = 11


def _sqnorm_rows(a8):
    return (a8[:, 0:1] * a8[:, 0:1] + a8[:, 1:2] * a8[:, 1:2]
            + a8[:, 2:3] * a8[:, 2:3])


def _sqnorm_cols(bt):
    return (bt[0:1, :] * bt[0:1, :] + bt[1:2, :] * bt[1:2, :]
            + bt[2:3, :] * bt[2:3, :])


def _row_block_size(n):
    return n if n < 512 else 512


def _loss_kernel(coarse_ref, fine1_ref, fine2_ref, partial_ref,
                 gt_t_ref, f2_t_ref, out_ref):
    gt_t = gt_t_ref[0]
    f2_t = f2_t_ref[0]
    gt_bfm2 = (gt_t * -2.0).astype(jnp.bfloat16)
    f2_bfm2 = (f2_t * -2.0).astype(jnp.bfloat16)
    b2_gt = _sqnorm_cols(gt_t)
    b2_f2 = _sqnorm_cols(f2_t)
    m_gt = gt_t.shape[1]
    m_f2 = f2_t.shape[1]

    def chamfer(a_ref, bt_bfm2, b2, m):
        n = a_ref.shape[1]
        r = _row_block_size(n)

        def body(i, carry):
            sv, colmin = carry
            a8 = a_ref[0, pl.ds(i * r, r), :]
            a2 = _sqnorm_rows(a8)
            ab2 = jnp.dot(a8.astype(jnp.bfloat16), bt_bfm2,
                          preferred_element_type=jnp.float32)
            e = b2 + ab2
            rmin = jnp.min(e, axis=1, keepdims=True) + a2
            sv = sv + jnp.maximum(rmin, 0.0)
            colmin = jnp.minimum(colmin, jnp.min(a2 + e, axis=0))
            return sv, colmin

        init = (jnp.zeros((r, 1), jnp.float32),
                jnp.full((m,), 1e30, jnp.float32))
        sv, colmin = jax.lax.fori_loop(0, n // r, body, init)
        return jnp.sum(sv), jnp.sum(jnp.maximum(colmin, 0.0))

    s_c_row, s_c_col = chamfer(coarse_ref, gt_bfm2, b2_gt, m_gt)
    s_f1_row, s_f1_col = chamfer(fine1_ref, gt_bfm2, b2_gt, m_gt)
    s_f2_row, s_f2_col = chamfer(fine2_ref, gt_bfm2, b2_gt, m_gt)

    n_p = partial_ref.shape[1]
    r_p = _row_block_size(n_p)

    def pbody(i, sv):
        a8 = partial_ref[0, pl.ds(i * r_p, r_p), :]
        a2 = _sqnorm_rows(a8)
        ab2 = jnp.dot(a8.astype(jnp.bfloat16), f2_bfm2,
                      preferred_element_type=jnp.float32)
        rmin = jnp.min(b2_f2 + ab2, axis=1, keepdims=True) + a2
        return sv + jnp.sqrt(jnp.maximum(rmin, 0.0))

    s_partial = jnp.sum(jax.lax.fori_loop(
        0, n_p // r_p, pbody, jnp.zeros((r_p, 1), jnp.float32)))

    b_aug = jnp.concatenate(
        [-2.0 * f2_t, jnp.ones((1, m_f2), jnp.float32), b2_f2,
         jnp.zeros((6, m_f2), jnp.float32)], axis=0)
    b_hi = b_aug.astype(jnp.bfloat16)
    b_lo = (b_aug - b_hi.astype(jnp.float32)).astype(jnp.bfloat16)
    n_f = fine2_ref.shape[1]
    r_f = _row_block_size(n_f)
    kf = float(_K1)

    def sbody(i, sv):
        a8 = fine2_ref[0, pl.ds(i * r_f, r_f), :]
        a2 = _sqnorm_rows(a8)
        ab2 = jnp.dot(a8.astype(jnp.bfloat16), f2_bfm2,
                      preferred_element_type=jnp.float32)
        keyb = (a2 + (b2_f2 + ab2)).astype(jnp.bfloat16)
        a_aug = jnp.concatenate(
            [a8, a2, jnp.ones((r_f, 1), jnp.float32),
             jnp.zeros((r_f, 6), jnp.float32)], axis=1)
        a_hi = a_aug.astype(jnp.bfloat16)
        a_lo = (a_aug - a_hi.astype(jnp.float32)).astype(jnp.bfloat16)
        val_raw = (jnp.dot(a_hi, b_hi, preferred_element_type=jnp.float32)
                   + (jnp.dot(a_hi, b_lo, preferred_element_type=jnp.float32)
                      + jnp.dot(a_lo, b_hi,
                                preferred_element_type=jnp.float32)))
        val = val_raw

        one_b = jnp.bfloat16(1.0)
        zero_b = jnp.bfloat16(0.0)
        big_b = jnp.bfloat16(1e30)

        m1 = jnp.min(keyb, axis=1, keepdims=True)
        le = keyb <= m1
        cnt0 = jnp.sum(jnp.where(le, one_b, zero_b), axis=1, keepdims=True,
                       dtype=jnp.bfloat16).astype(jnp.float32)
        krem = kf - jnp.minimum(cnt0, kf)
        tau = m1
        c_lt = jnp.zeros((r_f, 1), jnp.float32)
        c_eq = cnt0
        k = jnp.where(le, big_b, keyb)
        for p in range(_K1 - 1):
            m = jnp.min(k, axis=1, keepdims=True)
            le = k <= m
            cnt = jnp.sum(jnp.where(le, one_b, zero_b), axis=1, keepdims=True,
                          dtype=jnp.bfloat16).astype(jnp.float32)
            take = jnp.minimum(cnt, krem)
            sel_p = take > 0.0
            tau = jnp.where(sel_p, m, tau)
            c_lt = jnp.where(sel_p, kf - krem, c_lt)
            c_eq = jnp.where(sel_p, cnt, c_eq)
            krem = krem - take
            if p < _K1 - 2:
                k = jnp.where(le, big_b, k)

        frac = ((kf - c_lt) / c_eq).astype(jnp.bfloat16)
        beta = (1.0 / cnt0).astype(jnp.bfloat16)
        w = jnp.where(keyb < tau, one_b, zero_b)
        w = w + jnp.where(keyb == tau, frac, zero_b)
        w = w - jnp.where(keyb == m1, beta, zero_b)
        picked = jnp.sum(w.astype(jnp.float32) * val, axis=1, keepdims=True)
        return sv + picked

    s_smooth = jnp.sum(jax.lax.fori_loop(
        0, n_f // r_f, sbody, jnp.zeros((r_f, 1), jnp.float32)))

    lane = jax.lax.broadcasted_iota(jnp.int32, (1, 128), 1)
    vals = [s_c_row, s_c_col, s_f1_row, s_f1_col,
            s_f2_row, s_f2_col, s_partial, s_smooth]
    out = jnp.zeros((1, 128), jnp.float32)
    for j, v in enumerate(vals):
        out = jnp.where(lane == j, v, out)
    out_ref[...] = out[None]


def kernel(partial, coarse, fine1, fine2, gt):
    b, n_partial, _ = partial.shape
    n_coarse = coarse.shape[1]
    n_fine1 = fine1.shape[1]
    n_fine2 = fine2.shape[1]
    n_gt = gt.shape[1]

    def pad_rows(x):
        return jnp.concatenate(
            [x, jnp.zeros((b, x.shape[1], 5), x.dtype)], axis=2)

    def pad_t(x):
        xt = jnp.transpose(x, (0, 2, 1))
        return jnp.concatenate(
            [xt, jnp.zeros((b, 5, x.shape[1]), x.dtype)], axis=1)

    spec3 = lambda n: pl.BlockSpec((1, n, 8), lambda i: (i, 0, 0))
    spect = lambda n: pl.BlockSpec((1, 8, n), lambda i: (i, 0, 0))

    sums = pl.pallas_call(
        _loss_kernel,
        grid=(b,),
        in_specs=[spec3(n_coarse), spec3(n_fine1), spec3(n_fine2),
                  spec3(n_partial), spect(n_gt), spect(n_fine2)],
        out_specs=pl.BlockSpec((1, 1, 128), lambda i: (i, 0, 0)),
        out_shape=jax.ShapeDtypeStruct((b, 1, 128), jnp.float32),
    )(pad_rows(coarse), pad_rows(fine1), pad_rows(fine2), pad_rows(partial),
      pad_t(gt), pad_t(fine2))
    sums = sums[:, 0, :]

    cd_coarse = jnp.mean(sums[:, 0] / n_coarse + sums[:, 1] / n_gt)
    cd_fine1 = jnp.mean(sums[:, 2] / n_fine1 + sums[:, 3] / n_gt)
    cd_fine2 = jnp.mean(sums[:, 4] / n_fine2 + sums[:, 5] / n_gt)
    partial_loss = jnp.mean(sums[:, 6]) / n_partial
    smooth_loss = jnp.mean(sums[:, 7]) / (n_fine2 * (_K1 - 1))
    total = (cd_coarse + cd_fine1 + cd_fine2 +
             0.5 * partial_loss + 0.1 * smooth_loss)
    return (total, cd_coarse, cd_fine1, cd_fine2, partial_loss, smooth_loss)
